# Initial kernel scaffold; baseline (speedup 1.0000x reference)
import jax
import jax.numpy as jnp
from jax.experimental import pallas as pl


def kernel(class_logits, box_regression, proposals):
    raise NotImplementedError("write your pallas kernel here")


# speedup vs baseline: 1.0042x; 1.0042x over previous
import math

import jax
import jax.numpy as jnp
from jax.experimental import pallas as pl

_SCORE_THRESH = 0.05
_NMS_THRESH = 0.5
_DETS_PER_IMG = 100
_M_CAND = 500
_IMG_H = 800.0
_IMG_W = 1333.0
_CLIP = math.log(1000.0 / 16.0)


def _dense_body(logits_ref, reg_ref, prop_ref,
                probs_ref, x1_ref, y1_ref, x2_ref, y2_ref):
    lg = logits_ref[...]
    m = jnp.max(lg, axis=1, keepdims=True)
    e = jnp.exp(lg - m)
    probs_ref[...] = e / jnp.sum(e, axis=1, keepdims=True)

    p = prop_ref[...]
    w = p[:, 2:3] - p[:, 0:1] + 1.0
    h = p[:, 3:4] - p[:, 1:2] + 1.0
    cx = p[:, 0:1] + 0.5 * w
    cy = p[:, 1:2] + 0.5 * h

    dx = reg_ref[0] / 10.0
    dy = reg_ref[1] / 10.0
    dw = jnp.minimum(reg_ref[2] / 5.0, _CLIP)
    dh = jnp.minimum(reg_ref[3] / 5.0, _CLIP)

    pcx = dx * w + cx
    pcy = dy * h + cy
    pw = jnp.exp(dw) * w
    ph = jnp.exp(dh) * h

    x1_ref[...] = jnp.clip(pcx - 0.5 * pw, 0.0, _IMG_W - 1.0)
    y1_ref[...] = jnp.clip(pcy - 0.5 * ph, 0.0, _IMG_H - 1.0)
    x2_ref[...] = jnp.clip(pcx + 0.5 * pw - 1.0, 0.0, _IMG_W - 1.0)
    y2_ref[...] = jnp.clip(pcy + 0.5 * ph - 1.0, 0.0, _IMG_H - 1.0)


def _dense_stage(class_logits, reg_planes, proposals, block=2000):
    N, C = class_logits.shape
    grid = (N // block,)
    out = jax.ShapeDtypeStruct((N, C), jnp.float32)
    return pl.pallas_call(
        _dense_body,
        grid=grid,
        in_specs=[
            pl.BlockSpec((block, C), lambda i: (i, 0)),
            pl.BlockSpec((4, block, C), lambda i: (0, i, 0)),
            pl.BlockSpec((block, 4), lambda i: (i, 0)),
        ],
        out_specs=[pl.BlockSpec((block, C), lambda i: (i, 0))] * 5,
        out_shape=[out] * 5,
    )(class_logits, reg_planes, proposals)


def _iou_matrix(b):
    area = (b[:, 2] - b[:, 0] + 1.0) * (b[:, 3] - b[:, 1] + 1.0)
    xx1 = jnp.maximum(b[:, None, 0], b[None, :, 0])
    yy1 = jnp.maximum(b[:, None, 1], b[None, :, 1])
    xx2 = jnp.minimum(b[:, None, 2], b[None, :, 2])
    yy2 = jnp.minimum(b[:, None, 3], b[None, :, 3])
    w = jnp.maximum(xx2 - xx1 + 1.0, 0.0)
    h = jnp.maximum(yy2 - yy1 + 1.0, 0.0)
    inter = w * h
    return inter / (area[:, None] + area[None, :] - inter)


def _nms_keep(iou, valid):
    M = iou.shape[0]
    ar = jnp.arange(M)
    def body(keep, i):
        col = iou[:, i]
        sup = jnp.any(keep & (col > _NMS_THRESH) & (ar < i))
        keep = keep.at[i].set(valid[i] & jnp.logical_not(sup))
        return keep, None
    keep, _ = jax.lax.scan(body, jnp.zeros((M,), dtype=bool), jnp.arange(M))
    return keep


def _per_class(scores_j, boxes_j):
    masked = jnp.where(scores_j > _SCORE_THRESH, scores_j, -jnp.inf)
    top_scores, idx = jax.lax.top_k(masked, _M_CAND)
    cand_valid = top_scores > _SCORE_THRESH
    cand_boxes = boxes_j[idx]
    iou = _iou_matrix(cand_boxes)
    keep = _nms_keep(iou, cand_valid)
    return top_scores, cand_boxes, keep


def kernel(class_logits, box_regression, proposals):
    N, C = class_logits.shape
    reg_planes = box_regression.reshape(N, C, 4).transpose(2, 0, 1)
    probs, bx1, by1, bx2, by2 = _dense_stage(class_logits, reg_planes, proposals)

    scores_T = probs.T[1:]
    boxes = jnp.stack([bx1, by1, bx2, by2], axis=-1)
    boxes_T = jnp.transpose(boxes, (1, 0, 2))[1:]

    top_scores, cand_boxes, keep = jax.vmap(_per_class)(scores_T, boxes_T)
    flat_scores = jnp.where(keep, top_scores, -jnp.inf).reshape(-1)
    flat_boxes = cand_boxes.reshape(-1, 4)
    flat_labels = jnp.broadcast_to(
        jnp.arange(1, C, dtype=jnp.int32)[:, None], (C - 1, _M_CAND)).reshape(-1)
    final_scores, fidx = jax.lax.top_k(flat_scores, _DETS_PER_IMG)
    final_boxes = flat_boxes[fidx]
    final_labels = flat_labels[fidx]
    return final_scores, final_boxes, final_labels


# speedup vs baseline: 2.5434x; 2.5327x over previous
import math

import jax
import jax.numpy as jnp
from jax.experimental import pallas as pl

_SCORE_THRESH = 0.05
_NMS_THRESH = 0.5
_DETS_PER_IMG = 100
_M_CAND = 500
_IMG_H = 800.0
_IMG_W = 1333.0
_CLIP = math.log(1000.0 / 16.0)


def _dense_body(logits_ref, reg_ref, prop_ref,
                probs_ref, x1_ref, y1_ref, x2_ref, y2_ref):
    lg = logits_ref[...]
    m = jnp.max(lg, axis=1, keepdims=True)
    e = jnp.exp(lg - m)
    probs_ref[...] = e / jnp.sum(e, axis=1, keepdims=True)

    p = prop_ref[...]
    w = p[:, 2:3] - p[:, 0:1] + 1.0
    h = p[:, 3:4] - p[:, 1:2] + 1.0
    cx = p[:, 0:1] + 0.5 * w
    cy = p[:, 1:2] + 0.5 * h

    dx = reg_ref[0] / 10.0
    dy = reg_ref[1] / 10.0
    dw = jnp.minimum(reg_ref[2] / 5.0, _CLIP)
    dh = jnp.minimum(reg_ref[3] / 5.0, _CLIP)

    pcx = dx * w + cx
    pcy = dy * h + cy
    pw = jnp.exp(dw) * w
    ph = jnp.exp(dh) * h

    x1_ref[...] = jnp.clip(pcx - 0.5 * pw, 0.0, _IMG_W - 1.0)
    y1_ref[...] = jnp.clip(pcy - 0.5 * ph, 0.0, _IMG_H - 1.0)
    x2_ref[...] = jnp.clip(pcx + 0.5 * pw - 1.0, 0.0, _IMG_W - 1.0)
    y2_ref[...] = jnp.clip(pcy + 0.5 * ph - 1.0, 0.0, _IMG_H - 1.0)


def _dense_stage(class_logits, reg_planes, proposals, block=2000):
    N, C = class_logits.shape
    grid = (N // block,)
    out = jax.ShapeDtypeStruct((N, C), jnp.float32)
    return pl.pallas_call(
        _dense_body,
        grid=grid,
        in_specs=[
            pl.BlockSpec((block, C), lambda i: (i, 0)),
            pl.BlockSpec((4, block, C), lambda i: (0, i, 0)),
            pl.BlockSpec((block, 4), lambda i: (i, 0)),
        ],
        out_specs=[pl.BlockSpec((block, C), lambda i: (i, 0))] * 5,
        out_shape=[out] * 5,
    )(class_logits, reg_planes, proposals)


def _nms_body(x1_ref, y1_ref, x2_ref, y2_ref, valid_ref, keep_ref):
    x1 = x1_ref[...]
    y1 = y1_ref[...]
    x2 = x2_ref[...]
    y2 = y2_ref[...]
    area = (x2 - x1 + 1.0) * (y2 - y1 + 1.0)
    keep_ref[...] = jnp.zeros_like(x1)

    def body(i, _):
        bx1 = x1_ref[pl.ds(i, 1), :]
        by1 = y1_ref[pl.ds(i, 1), :]
        bx2 = x2_ref[pl.ds(i, 1), :]
        by2 = y2_ref[pl.ds(i, 1), :]
        barea = (bx2 - bx1 + 1.0) * (by2 - by1 + 1.0)
        w = jnp.maximum(jnp.minimum(bx2, x2) - jnp.maximum(bx1, x1) + 1.0, 0.0)
        h = jnp.maximum(jnp.minimum(by2, y2) - jnp.maximum(by1, y1) + 1.0, 0.0)
        inter = w * h
        sup = inter > _NMS_THRESH * (area + barea - inter)
        kp = keep_ref[...]
        hit = jnp.max(jnp.where(sup, kp, 0.0), axis=0, keepdims=True)
        keep_ref[pl.ds(i, 1), :] = valid_ref[pl.ds(i, 1), :] * (1.0 - hit)
        return 0

    jax.lax.fori_loop(0, _M_CAND, body, 0)


def _nms_stage(planes, valid):
    M, CL = valid.shape
    spec = pl.BlockSpec((M, CL), lambda: (0, 0))
    return pl.pallas_call(
        _nms_body,
        in_specs=[pl.BlockSpec((M, CL), lambda: (0, 0))] * 5,
        out_specs=spec,
        out_shape=jax.ShapeDtypeStruct((M, CL), jnp.float32),
    )(planes[0], planes[1], planes[2], planes[3], valid)


def kernel(class_logits, box_regression, proposals):
    N, C = class_logits.shape
    reg_planes = box_regression.reshape(N, C, 4).transpose(2, 0, 1)
    probs, bx1, by1, bx2, by2 = _dense_stage(class_logits, reg_planes, proposals)

    scores_T = probs.T[1:]
    boxes = jnp.stack([bx1, by1, bx2, by2], axis=-1)
    boxes_T = jnp.transpose(boxes, (1, 0, 2))[1:]

    masked = jnp.where(scores_T > _SCORE_THRESH, scores_T, -jnp.inf)
    top_scores, idx = jax.lax.top_k(masked, _M_CAND)
    cand_valid = (top_scores > _SCORE_THRESH).astype(jnp.float32)
    cand_boxes = jnp.take_along_axis(boxes_T, idx[..., None], axis=1)

    planes = cand_boxes.transpose(2, 1, 0)
    keep_f = _nms_stage(planes, cand_valid.T)
    keep = keep_f.T > 0.5

    flat_scores = jnp.where(keep, top_scores, -jnp.inf).reshape(-1)
    flat_boxes = cand_boxes.reshape(-1, 4)
    flat_labels = jnp.broadcast_to(
        jnp.arange(1, C, dtype=jnp.int32)[:, None], (C - 1, _M_CAND)).reshape(-1)
    final_scores, fidx = jax.lax.top_k(flat_scores, _DETS_PER_IMG)
    final_boxes = flat_boxes[fidx]
    final_labels = flat_labels[fidx]
    return final_scores, final_boxes, final_labels
